# Initial kernel scaffold; baseline (speedup 1.0000x reference)
import jax
import jax.numpy as jnp
from jax.experimental import pallas as pl


def kernel(idx, table):
    raise NotImplementedError("write your pallas kernel here")


# speedup vs baseline: 7.9745x; 7.9745x over previous
import functools

import jax
import jax.numpy as jnp
from jax import lax
from jax.experimental import pallas as pl
from jax.experimental.pallas import tpu as pltpu
from jax.experimental.pallas import tpu_sc as plsc

CYCLE = 3
D = 128
BATCH = 16384
HIST = 200
NTOT = BATCH * HIST

NC = 2
NS = 16
NW = NC * NS
PER_W = NTOT // NW

CH = 128
BSUB = 8
BLK_ROWS = BSUB * CH
NBLK = PER_W // BLK_ROWS
IDX_ROWS_W = PER_W // CH


def _body(idx_hbm, table_hbm, out_hbm, tab_sh, idxb, rows, gsem):
    cid = lax.axis_index("c")
    sid = lax.axis_index("s")
    wid = sid * NC + cid

    @pl.when(sid == 0)
    def _():
        pltpu.sync_copy(table_hbm, tab_sh)

    plsc.subcore_barrier()

    idx_row0 = wid * IDX_ROWS_W
    out_row0 = wid * PER_W

    def blk(g, carry):
        pltpu.sync_copy(idx_hbm.at[pl.ds(idx_row0 + g * BSUB, BSUB)], idxb)
        for j in range(BSUB):
            for i in range(CH // 16):
                v = idxb[j, pl.ds(i * 16, 16)]
                idxb[j, pl.ds(i * 16, 16)] = jnp.mod(v, CYCLE)
        for j in range(BSUB):
            pltpu.async_copy(tab_sh.at[idxb.at[j]], rows, gsem).wait()
            pltpu.sync_copy(
                rows, out_hbm.at[pl.ds(out_row0 + (g * BSUB + j) * CH, CH)]
            )
        return carry

    lax.fori_loop(0, NBLK, blk, 0)


@jax.jit
def _run(idx2, table):
    mesh = plsc.VectorSubcoreMesh(core_axis_name="c", subcore_axis_name="s")
    return pl.kernel(
        _body,
        out_type=jax.ShapeDtypeStruct((NTOT, D), jnp.float32),
        mesh=mesh,
        scratch_types=[
            pltpu.VMEM_SHARED((CYCLE, D), jnp.float32),
            pltpu.VMEM((BSUB, CH), jnp.int32),
            pltpu.VMEM((CH, D), jnp.float32),
            pltpu.SemaphoreType.DMA,
        ],
    )(idx2, table)


def kernel(idx, table):
    idx2 = idx.reshape(NTOT // CH, CH)
    out = _run(idx2, table)
    return out.reshape(BATCH, HIST, D)


# speedup vs baseline: 14.2695x; 1.7894x over previous
import functools

import jax
import jax.numpy as jnp
from jax import lax
from jax.experimental import pallas as pl
from jax.experimental.pallas import tpu as pltpu
from jax.experimental.pallas import tpu_sc as plsc

CYCLE = 3
D = 128
BATCH = 16384
HIST = 200
NTOT = BATCH * HIST

NC = 2
NS = 16
NW = NC * NS
PER_W = NTOT // NW

CH = 128
BSUB = 8
BLK = BSUB * CH
NBLK = PER_W // BLK
IDX_ROWS_W = PER_W // CH
RING = 4


def _body(idx_hbm, table_hbm, out_hbm, tab_sh, idxb, rows,
          is0, is1, gs0, gs1, gs2, gs3, os0, os1, os2, os3):
    isem = [is0, is1]
    gsem = [gs0, gs1, gs2, gs3]
    osem = [os0, os1, os2, os3]

    cid = lax.axis_index("c")
    sid = lax.axis_index("s")
    wid = sid * NC + cid

    @pl.when(sid == 0)
    def _():
        pltpu.sync_copy(table_hbm, tab_sh)

    plsc.subcore_barrier()

    idx_row0 = wid * PER_W
    out_row0 = wid * PER_W

    def idx_src(g):
        return idx_hbm.at[pl.ds(idx_row0 + g * BLK, BLK)]

    def out_dst(gidx):
        return out_hbm.at[pl.ds(out_row0 + gidx * CH, CH)]

    def wait_idx(bb):
        pltpu.make_async_copy(idx_src(0), idxb.at[bb], isem[bb]).wait()

    def wait_gat(p):
        pltpu.make_async_copy(
            tab_sh.at[idxb.at[0, pl.ds(0, CH)]], rows.at[p], gsem[p]
        ).wait()

    def wait_out(p):
        pltpu.make_async_copy(rows.at[p], out_dst(0), osem[p]).wait()

    pltpu.async_copy(idx_src(0), idxb.at[0], isem[0])

    def blk2(g2, carry):
        for bb in range(2):
            g = g2 * 2 + bb
            wait_idx(bb)

            def wrap(i, c):
                v = idxb[bb, pl.ds(i * 16, 16)]
                idxb[bb, pl.ds(i * 16, 16)] = jnp.mod(v, CYCLE)
                return c

            lax.fori_loop(0, BLK // 16, wrap, 0)

            for j in range(BSUB):
                p = j % RING
                pm = (j - 1) % RING
                if bb == 0 and j < RING:
                    @pl.when(g2 > 0)
                    def _():
                        wait_out(p)
                else:
                    wait_out(p)
                pltpu.async_copy(
                    tab_sh.at[idxb.at[bb, pl.ds(j * CH, CH)]],
                    rows.at[p], gsem[p],
                )
                if j == 0:
                    @pl.when(g > 0)
                    def _():
                        wait_gat(pm)
                        pltpu.async_copy(
                            rows.at[pm], out_dst(g * BSUB - 1), osem[pm]
                        )
                else:
                    wait_gat(pm)
                    pltpu.async_copy(
                        rows.at[pm], out_dst(g * BSUB + j - 1), osem[pm]
                    )
                if j == 0:
                    @pl.when(g < NBLK - 1)
                    def _():
                        pltpu.async_copy(
                            idx_src(g + 1), idxb.at[1 - bb], isem[1 - bb]
                        )
        return carry

    lax.fori_loop(0, NBLK // 2, blk2, 0)

    last = NBLK * BSUB - 1
    pl_last = (BSUB - 1) % RING
    wait_gat(pl_last)
    pltpu.async_copy(rows.at[pl_last], out_dst(last), osem[pl_last])
    for p in range(RING):
        wait_out(p)


@jax.jit
def _run(idx2, table):
    mesh = plsc.VectorSubcoreMesh(core_axis_name="c", subcore_axis_name="s")
    return pl.kernel(
        _body,
        out_type=jax.ShapeDtypeStruct((NTOT, D), jnp.float32),
        mesh=mesh,
        scratch_types=[
            pltpu.VMEM_SHARED((CYCLE, D), jnp.float32),
            pltpu.VMEM((2, BLK), jnp.int32),
            pltpu.VMEM((RING, CH, D), jnp.float32),
            pltpu.SemaphoreType.DMA,
            pltpu.SemaphoreType.DMA,
            pltpu.SemaphoreType.DMA,
            pltpu.SemaphoreType.DMA,
            pltpu.SemaphoreType.DMA,
            pltpu.SemaphoreType.DMA,
            pltpu.SemaphoreType.DMA,
            pltpu.SemaphoreType.DMA,
            pltpu.SemaphoreType.DMA,
            pltpu.SemaphoreType.DMA,
        ],
    )(idx2, table)


def kernel(idx, table):
    out = _run(idx.reshape(NTOT), table)
    return out.reshape(BATCH, HIST, D)
